# baseline (device time: 17056 ns/iter reference)
import jax
import jax.numpy as jnp
from jax import lax
from jax.experimental import pallas as pl
from jax.experimental.pallas import tpu as pltpu

N_DEV = 4
N_SEG = 8


def kernel(x, w_mat):
    x = x.astype(jnp.bfloat16)
    w_mat = w_mat.astype(jnp.bfloat16)
    m_per, k = x.shape
    _, n_per = w_mat.shape
    seg = m_per // N_SEG
    half = m_per // 2

    def body(x_ref, w_ref, out_ref,
             from_left, from_right, diag,
             s_r1r, s_r1l, r_r1r, r_r1l,
             s_r2r, s_r2l, r_r2r, r_r2l):
        my_pos = lax.axis_index("i")
        left = (my_pos - 1) % N_DEV
        right = (my_pos + 1) % N_DEV

        barrier_sem = pltpu.get_barrier_semaphore()
        for nbr in [left, right]:
            pl.semaphore_signal(
                barrier_sem, inc=1,
                device_id=(nbr,), device_id_type=pl.DeviceIdType.MESH,
            )
        pl.semaphore_wait(barrier_sem, 2)

        def seg_copy(src, dst, s, send_sem, recv_sem, idx, dev):
            return pltpu.make_async_remote_copy(
                src_ref=src.at[pl.ds(s * seg, seg)],
                dst_ref=dst.at[pl.ds(s * seg, seg)],
                send_sem=send_sem.at[idx], recv_sem=recv_sem.at[idx],
                device_id=(dev,), device_id_type=pl.DeviceIdType.MESH,
            )

        r1r = [seg_copy(x_ref, from_left, s, s_r1r, r_r1r, s, right)
               for s in range(N_SEG)]
        r1l = [seg_copy(x_ref, from_right, s, s_r1l, r_r1l, s, left)
               for s in range(N_SEG)]
        for s in range(N_SEG):
            r1r[s].start()
            r1l[N_SEG - 1 - s].start()

        w = w_ref[...]
        acc = jnp.dot(x_ref[...], w, preferred_element_type=jnp.float32)
        out_ref[pl.ds(my_pos * m_per, m_per), :] = jnp.maximum(acc, 0.0)

        n_fwd = N_SEG // 2
        r2r = [seg_copy(from_left, diag, j, s_r2r, r_r2r, j, right)
               for j in range(n_fwd)]
        r2l = [seg_copy(from_right, diag, N_SEG - 1 - j, s_r2l, r_r2l, j, left)
               for j in range(n_fwd)]

        for j in range(n_fwd):
            r1r[j].wait_recv()
            r2r[j].start()
            r1l[N_SEG - 1 - j].wait_recv()
            r2l[j].start()

        for s in range(n_fwd, N_SEG):
            r1r[s].wait_recv()
        acc = jnp.dot(from_left[...], w, preferred_element_type=jnp.float32)
        out_ref[pl.ds(left * m_per, m_per), :] = jnp.maximum(acc, 0.0)

        for s in reversed(range(n_fwd)):
            r1l[s].wait_recv()
        acc = jnp.dot(from_right[...], w, preferred_element_type=jnp.float32)
        out_ref[pl.ds(right * m_per, m_per), :] = jnp.maximum(acc, 0.0)

        diag_pos = (my_pos + 2) % N_DEV
        for j in range(n_fwd):
            r2r[j].wait_recv()
        acc = jnp.dot(diag[pl.ds(0, half)], w, preferred_element_type=jnp.float32)
        out_ref[pl.ds(diag_pos * m_per, half), :] = jnp.maximum(acc, 0.0)

        for j in range(n_fwd):
            r2l[j].wait_recv()
        acc = jnp.dot(diag[pl.ds(half, half)], w, preferred_element_type=jnp.float32)
        out_ref[pl.ds(diag_pos * m_per + half, half), :] = jnp.maximum(acc, 0.0)

        for d in r1r + r1l + r2r + r2l:
            d.wait_send()

    return pl.pallas_call(
        body,
        out_shape=jax.ShapeDtypeStruct((N_DEV * m_per, n_per), jnp.float32),
        in_specs=[
            pl.BlockSpec(memory_space=pltpu.VMEM),
            pl.BlockSpec(memory_space=pltpu.VMEM),
        ],
        out_specs=pl.BlockSpec(memory_space=pltpu.VMEM),
        scratch_shapes=[
            pltpu.VMEM((m_per, k), jnp.bfloat16),
            pltpu.VMEM((m_per, k), jnp.bfloat16),
            pltpu.VMEM((m_per, k), jnp.bfloat16),
            pltpu.SemaphoreType.DMA((N_SEG,)),
            pltpu.SemaphoreType.DMA((N_SEG,)),
            pltpu.SemaphoreType.DMA((N_SEG,)),
            pltpu.SemaphoreType.DMA((N_SEG,)),
            pltpu.SemaphoreType.DMA((N_SEG // 2,)),
            pltpu.SemaphoreType.DMA((N_SEG // 2,)),
            pltpu.SemaphoreType.DMA((N_SEG // 2,)),
            pltpu.SemaphoreType.DMA((N_SEG // 2,)),
        ],
        compiler_params=pltpu.CompilerParams(collective_id=0),
    )(x, w_mat)


# device time: 15685 ns/iter; 1.0874x vs baseline; 1.0874x over previous
import jax
import jax.numpy as jnp
from jax import lax
from jax.experimental import pallas as pl
from jax.experimental.pallas import tpu as pltpu

N_DEV = 4
N_SEG = 4


def kernel(x, w_mat):
    x = x.astype(jnp.bfloat16)
    w_mat = w_mat.astype(jnp.bfloat16)
    m_per, k = x.shape
    _, n_per = w_mat.shape
    seg = m_per // N_SEG
    half = m_per // 2

    def body(x_ref, w_ref, out_ref,
             from_left, from_right, diag,
             s_r1r, s_r1l, r_r1r, r_r1l,
             s_r2r, s_r2l, r_r2r, r_r2l):
        my_pos = lax.axis_index("i")
        left = (my_pos - 1) % N_DEV
        right = (my_pos + 1) % N_DEV

        barrier_sem = pltpu.get_barrier_semaphore()
        for nbr in [left, right]:
            pl.semaphore_signal(
                barrier_sem, inc=1,
                device_id=(nbr,), device_id_type=pl.DeviceIdType.MESH,
            )
        pl.semaphore_wait(barrier_sem, 2)

        def seg_copy(src, dst, s, send_sem, recv_sem, idx, dev):
            return pltpu.make_async_remote_copy(
                src_ref=src.at[pl.ds(s * seg, seg)],
                dst_ref=dst.at[pl.ds(s * seg, seg)],
                send_sem=send_sem.at[idx], recv_sem=recv_sem.at[idx],
                device_id=(dev,), device_id_type=pl.DeviceIdType.MESH,
            )

        r1r = [seg_copy(x_ref, from_left, s, s_r1r, r_r1r, s, right)
               for s in range(N_SEG)]
        r1l = [seg_copy(x_ref, from_right, s, s_r1l, r_r1l, s, left)
               for s in range(N_SEG)]
        for s in range(N_SEG):
            r1r[s].start()
            r1l[N_SEG - 1 - s].start()

        w = w_ref[...]
        acc = jnp.dot(x_ref[...], w, preferred_element_type=jnp.float32)
        out_ref[pl.ds(my_pos * m_per, m_per), :] = jnp.maximum(acc, 0.0)

        n_fwd = N_SEG // 2
        r2r = [seg_copy(from_left, diag, j, s_r2r, r_r2r, j, right)
               for j in range(n_fwd)]
        r2l = [seg_copy(from_right, diag, N_SEG - 1 - j, s_r2l, r_r2l, j, left)
               for j in range(n_fwd)]

        for j in range(n_fwd):
            r1r[j].wait_recv()
            r2r[j].start()
            r1l[N_SEG - 1 - j].wait_recv()
            r2l[j].start()

        for s in range(n_fwd, N_SEG):
            r1r[s].wait_recv()
        acc = jnp.dot(from_left[...], w, preferred_element_type=jnp.float32)
        out_ref[pl.ds(left * m_per, m_per), :] = jnp.maximum(acc, 0.0)

        for s in reversed(range(n_fwd)):
            r1l[s].wait_recv()
        acc = jnp.dot(from_right[...], w, preferred_element_type=jnp.float32)
        out_ref[pl.ds(right * m_per, m_per), :] = jnp.maximum(acc, 0.0)

        diag_pos = (my_pos + 2) % N_DEV
        for j in range(n_fwd):
            r2r[j].wait_recv()
        acc = jnp.dot(diag[pl.ds(0, half)], w, preferred_element_type=jnp.float32)
        out_ref[pl.ds(diag_pos * m_per, half), :] = jnp.maximum(acc, 0.0)

        for j in range(n_fwd):
            r2l[j].wait_recv()
        acc = jnp.dot(diag[pl.ds(half, half)], w, preferred_element_type=jnp.float32)
        out_ref[pl.ds(diag_pos * m_per + half, half), :] = jnp.maximum(acc, 0.0)

        for d in r1r + r1l + r2r + r2l:
            d.wait_send()

    return pl.pallas_call(
        body,
        out_shape=jax.ShapeDtypeStruct((N_DEV * m_per, n_per), jnp.float32),
        in_specs=[
            pl.BlockSpec(memory_space=pltpu.VMEM),
            pl.BlockSpec(memory_space=pltpu.VMEM),
        ],
        out_specs=pl.BlockSpec(memory_space=pltpu.VMEM),
        scratch_shapes=[
            pltpu.VMEM((m_per, k), jnp.bfloat16),
            pltpu.VMEM((m_per, k), jnp.bfloat16),
            pltpu.VMEM((m_per, k), jnp.bfloat16),
            pltpu.SemaphoreType.DMA((N_SEG,)),
            pltpu.SemaphoreType.DMA((N_SEG,)),
            pltpu.SemaphoreType.DMA((N_SEG,)),
            pltpu.SemaphoreType.DMA((N_SEG,)),
            pltpu.SemaphoreType.DMA((N_SEG // 2,)),
            pltpu.SemaphoreType.DMA((N_SEG // 2,)),
            pltpu.SemaphoreType.DMA((N_SEG // 2,)),
            pltpu.SemaphoreType.DMA((N_SEG // 2,)),
        ],
        compiler_params=pltpu.CompilerParams(collective_id=0),
    )(x, w_mat)
